# baseline (speedup 1.0000x reference)
import functools

import jax
import jax.numpy as jnp
from jax.experimental import pallas as pl
from jax.experimental.pallas import tpu as pltpu

N_NODES = 10000
N_EDGES = 320000
LATENT = 64
GDIM = 8

E_PAD = 323584
BE = 4096
NBE = E_PAD // BE
BN = 2000
NBN = N_NODES // BN


def _relu(x):
    return jnp.maximum(x, 0.0)


def _ln(x, scale, bias, eps=1e-6):
    m = jnp.mean(x, axis=-1, keepdims=True)
    xc = x - m
    v = jnp.mean(xc * xc, axis=-1, keepdims=True)
    return xc * jax.lax.rsqrt(v + eps) * scale + bias


def _enc_body(nodes_ref, wenc_ref, benc_ref, wproj_ref, gmat_ref, wg_ref,
              brows_ref, n_ref, ps_ref, pr_ref, cvec_ref):
    n = jnp.dot(nodes_ref[...], wenc_ref[...],
                preferred_element_type=jnp.float32) + benc_ref[...]
    n_ref[...] = n
    p = jnp.dot(n, wproj_ref[...], preferred_element_type=jnp.float32)
    ps_ref[...] = p[:, :LATENT]
    pr_ref[...] = p[:, LATENT:]
    cvec_ref[...] = jnp.dot(gmat_ref[...], wg_ref[...],
                            preferred_element_type=jnp.float32) + brows_ref[...]


def _encoder_call(nodes, wenc, benc, wproj, gmat, wg, brows):
    full = lambda i: (0, 0)
    return pl.pallas_call(
        _enc_body,
        grid=(NBN,),
        in_specs=[
            pl.BlockSpec((BN, 128), lambda i: (i, 0)),
            pl.BlockSpec((128, LATENT), full),
            pl.BlockSpec((1, LATENT), full),
            pl.BlockSpec((LATENT, 2 * LATENT), full),
            pl.BlockSpec((4, 4 * GDIM), full),
            pl.BlockSpec((4 * GDIM, LATENT), full),
            pl.BlockSpec((4, LATENT), full),
        ],
        out_specs=[
            pl.BlockSpec((BN, LATENT), lambda i: (i, 0)),
            pl.BlockSpec((BN, LATENT), lambda i: (i, 0)),
            pl.BlockSpec((BN, LATENT), lambda i: (i, 0)),
            pl.BlockSpec((4, LATENT), full),
        ],
        out_shape=[
            jax.ShapeDtypeStruct((N_NODES, LATENT), jnp.float32),
            jax.ShapeDtypeStruct((N_NODES, LATENT), jnp.float32),
            jax.ShapeDtypeStruct((N_NODES, LATENT), jnp.float32),
            jax.ShapeDtypeStruct((4, LATENT), jnp.float32),
        ],
    )(nodes, wenc, benc, wproj, gmat, wg, brows)


def _edge_body(has_e, ce_row, gs_ref, gr_ref, e_ref, w1e_ref, w2_ref,
               b2_ref, w3_ref, b3_ref, ce_ref, out_ref):
    x = gs_ref[...] + gr_ref[...] + ce_ref[ce_row:ce_row + 1, :]
    if has_e:
        x = x + jnp.dot(e_ref[...], w1e_ref[...],
                        preferred_element_type=jnp.float32)
    h1 = _relu(x)
    h2 = _relu(jnp.dot(h1, w2_ref[...],
                       preferred_element_type=jnp.float32) + b2_ref[...])
    y = jnp.dot(h2, w3_ref[...],
                preferred_element_type=jnp.float32) + b3_ref[...]
    rows = pl.program_id(0) * BE + jax.lax.broadcasted_iota(
        jnp.int32, (BE, 1), 0)
    out_ref[...] = jnp.where(rows < N_EDGES, y, 0.0)


def _edge_call(gathered, e_prev, w1e, w2, b2, w3, b3, cvec, ce_row):
    has_e = e_prev is not None
    full = lambda i: (0, 0)
    in_specs = [
        pl.BlockSpec((BE, LATENT), lambda i: (i, 0)),
        pl.BlockSpec((BE, LATENT), lambda i: (i + NBE, 0)),
    ]
    args = [gathered, gathered]
    if has_e:
        in_specs += [pl.BlockSpec((BE, LATENT), lambda i: (i, 0)),
                     pl.BlockSpec((LATENT, LATENT), full)]
        args += [e_prev, w1e]
    in_specs += [
        pl.BlockSpec((LATENT, LATENT), full),
        pl.BlockSpec((1, LATENT), full),
        pl.BlockSpec((LATENT, LATENT), full),
        pl.BlockSpec((1, LATENT), full),
        pl.BlockSpec((4, LATENT), full),
    ]
    args += [w2, b2, w3, b3, cvec]
    body = functools.partial(_edge_body, has_e, ce_row)
    if not has_e:
        def body(gs, gr, w2r, b2r, w3r, b3r, cer, outr):
            _edge_body(False, ce_row, gs, gr, None, None, w2r, b2r, w3r,
                       b3r, cer, outr)
    return pl.pallas_call(
        body,
        grid=(NBE,),
        in_specs=in_specs,
        out_specs=pl.BlockSpec((BE, LATENT), lambda i: (i, 0)),
        out_shape=jax.ShapeDtypeStruct((E_PAD, LATENT), jnp.float32),
    )(*args)


def _node_body(final, cn_row, n_ref, r0_ref, r1_ref, a1_ref, bmat_ref,
               w2_ref, b2_ref, w3_ref, b3_ref, cn_ref, lns_ref, lnb_ref,
               wnext_ref, bnext_ref, *out_refs):
    n = n_ref[...]
    recv = r0_ref[...] + r1_ref[...]
    m1 = _relu(jnp.dot(n, a1_ref[...], preferred_element_type=jnp.float32)
               + jnp.dot(recv, bmat_ref[...],
                         preferred_element_type=jnp.float32)
               + cn_ref[cn_row:cn_row + 1, :])
    m2 = _relu(jnp.dot(m1, w2_ref[...],
                       preferred_element_type=jnp.float32) + b2_ref[...])
    nn = jnp.dot(m2, w3_ref[...],
                 preferred_element_type=jnp.float32) + b3_ref[...]
    y = _ln(nn + n, lns_ref[...], lnb_ref[...])
    if final:
        out_refs[0][...] = jnp.dot(
            y, wnext_ref[...], preferred_element_type=jnp.float32
        ) + bnext_ref[...]
    else:
        out_refs[0][...] = y
        p = jnp.dot(y, wnext_ref[...], preferred_element_type=jnp.float32)
        out_refs[1][...] = p[:, :LATENT]
        out_refs[2][...] = p[:, LATENT:]


def _node_call(final, n, r0, r1, a1, bmat, w2, b2, w3, b3, cvec, cn_row,
               lns, lnb, wnext, bnext):
    full = lambda i: (0, 0)
    blk = lambda i: (i, 0)
    next_cols = 128 if final else 2 * LATENT
    in_specs = [
        pl.BlockSpec((BN, LATENT), blk),
        pl.BlockSpec((BN, LATENT), blk),
        pl.BlockSpec((BN, LATENT), blk),
        pl.BlockSpec((LATENT, LATENT), full),
        pl.BlockSpec((LATENT, LATENT), full),
        pl.BlockSpec((LATENT, LATENT), full),
        pl.BlockSpec((1, LATENT), full),
        pl.BlockSpec((LATENT, LATENT), full),
        pl.BlockSpec((1, LATENT), full),
        pl.BlockSpec((4, LATENT), full),
        pl.BlockSpec((1, LATENT), full),
        pl.BlockSpec((1, LATENT), full),
        pl.BlockSpec((LATENT, next_cols), full),
        pl.BlockSpec((1, 128), full),
    ]
    if final:
        out_specs = pl.BlockSpec((BN, 128), blk)
        out_shape = jax.ShapeDtypeStruct((N_NODES, 128), jnp.float32)
    else:
        out_specs = [pl.BlockSpec((BN, LATENT), blk)] * 3
        out_shape = [jax.ShapeDtypeStruct((N_NODES, LATENT), jnp.float32)] * 3
    return pl.pallas_call(
        functools.partial(_node_body, final, cn_row),
        grid=(NBN,),
        in_specs=in_specs,
        out_specs=out_specs,
        out_shape=out_shape,
    )(n, r0, r1, a1, bmat, w2, b2, w3, b3, cvec, lns, lnb, wnext, bnext)


def _gather_rows(table, idx2):
    return jnp.take(table, idx2, axis=0)


def _segment_partials(e_new, receivers_p):
    s = jax.ops.segment_sum(e_new, receivers_p, num_segments=N_NODES)
    return s, jnp.zeros_like(s)


def kernel(nodes, senders, receivers, globals_, params):
    f32 = jnp.float32
    enc = params["encoder"]
    dec = params["decoder"]
    steps = params["steps"]

    pad = E_PAD - N_EDGES
    senders_p = jnp.concatenate([senders, jnp.zeros((pad,), senders.dtype)])
    receivers_p = jnp.concatenate(
        [receivers, jnp.zeros((pad,), receivers.dtype)])
    idx2 = jnp.concatenate([senders_p, receivers_p + N_NODES]).astype(
        jnp.int32)

    g0 = globals_[0].astype(f32)

    e0W1 = steps[0]["edge_mlp"][0]["W"]
    w1s0, w1r0, w1g0 = e0W1[:64], e0W1[64:128], e0W1[128:136]
    e1W1 = steps[1]["edge_mlp"][0]["W"]
    w1e1, w1s1, w1r1, w1g1 = (e1W1[:64], e1W1[64:128], e1W1[128:192],
                              e1W1[192:200])
    n0W1 = steps[0]["node_mlp"][0]["W"]
    a10, bm0, wgn0 = n0W1[:64], n0W1[64:128], n0W1[128:136]
    n1W1 = steps[1]["node_mlp"][0]["W"]
    a11, bm1, wgn1 = n1W1[:64], n1W1[64:128], n1W1[128:136]

    gmat = jnp.zeros((4, 4 * GDIM), f32)
    gmat = gmat.at[0, 0:8].set(g0)
    gmat = gmat.at[1, 8:16].set(g0)
    gmat = gmat.at[2, 16:24].set(2.0 * g0)
    gmat = gmat.at[3, 24:32].set(2.0 * g0)
    wg_all = jnp.concatenate([w1g0, wgn0, w1g1, wgn1], axis=0)
    brows = jnp.stack([
        steps[0]["edge_mlp"][0]["b"], steps[0]["node_mlp"][0]["b"],
        steps[1]["edge_mlp"][0]["b"], steps[1]["node_mlp"][0]["b"]])

    row = lambda b: b.reshape(1, -1)
    wproj0 = jnp.concatenate([w1s0, w1r0], axis=1)
    wproj1 = jnp.concatenate([w1s1, w1r1], axis=1)

    n0, ps0, pr0, cvec = _encoder_call(
        nodes, enc["W"], row(enc["b"]), wproj0, gmat, wg_all, brows)

    t0 = jnp.concatenate([ps0, pr0], axis=0)
    g0rows = _gather_rows(t0, idx2)
    em0 = steps[0]["edge_mlp"]
    e_new0 = _edge_call(g0rows, None, None, em0[1]["W"], row(em0[1]["b"]),
                        em0[2]["W"], row(em0[2]["b"]), cvec, 0)
    p0a, p0b = _segment_partials(e_new0, receivers_p)
    nm0 = steps[0]["node_mlp"]
    n1, ps1, pr1 = _node_call(
        False, n0, p0a, p0b, a10, bm0, nm0[1]["W"], row(nm0[1]["b"]),
        nm0[2]["W"], row(nm0[2]["b"]), cvec, 1,
        row(steps[0]["ln_scale"]), row(steps[0]["ln_bias"]),
        wproj1, jnp.zeros((1, 128), f32))

    t1 = jnp.concatenate([ps1, pr1], axis=0)
    g1rows = _gather_rows(t1, idx2)
    em1 = steps[1]["edge_mlp"]
    e_new1 = _edge_call(g1rows, e_new0, w1e1, em1[1]["W"], row(em1[1]["b"]),
                        em1[2]["W"], row(em1[2]["b"]), cvec, 2)
    p1a, p1b = _segment_partials(e_new1, receivers_p)
    nm1 = steps[1]["node_mlp"]
    out = _node_call(
        True, n1, p1a, p1b, a11, bm1, nm1[1]["W"], row(nm1[1]["b"]),
        nm1[2]["W"], row(nm1[2]["b"]), cvec, 3,
        row(steps[1]["ln_scale"]), row(steps[1]["ln_bias"]),
        dec["W"], row(dec["b"]))
    return out


# speedup vs baseline: 3.2492x; 3.2492x over previous
import functools

import jax
import jax.numpy as jnp
from jax.experimental import pallas as pl
from jax.experimental.pallas import tpu as pltpu
from jax.experimental.pallas import tpu_sc as plsc

N_NODES = 10000
N_EDGES = 320000
LATENT = 64
GDIM = 8

NC, NS = 2, 16
NW = NC * NS
CHUNK = 128
SLOTS = 4
E_PAD = 327680
BE = 4096
NBE = E_PAD // BE
BN = 2000
NBN = N_NODES // BN
N_ACC = 10240
ROWS_PER_TILE = N_ACC // NS


def _relu(x):
    return jnp.maximum(x, 0.0)


def _ln(x, scale, bias, eps=1e-6):
    m = jnp.mean(x, axis=-1, keepdims=True)
    xc = x - m
    v = jnp.mean(xc * xc, axis=-1, keepdims=True)
    return xc * jax.lax.rsqrt(v + eps) * scale + bias


def _enc_body(nodes_ref, wenc_ref, benc_ref, wproj_ref, gmat_ref, wg_ref,
              brows_ref, n_ref, ps_ref, pr_ref, cvec_ref):
    n = jnp.dot(nodes_ref[...], wenc_ref[...],
                preferred_element_type=jnp.float32) + benc_ref[...]
    n_ref[...] = n
    p = jnp.dot(n, wproj_ref[...], preferred_element_type=jnp.float32)
    ps_ref[...] = p[:, :LATENT]
    pr_ref[...] = p[:, LATENT:]
    cvec_ref[...] = jnp.dot(gmat_ref[...], wg_ref[...],
                            preferred_element_type=jnp.float32) + brows_ref[...]


def _encoder_call(nodes, wenc, benc, wproj, gmat, wg, brows):
    full = lambda i: (0, 0)
    return pl.pallas_call(
        _enc_body,
        grid=(NBN,),
        in_specs=[
            pl.BlockSpec((BN, 128), lambda i: (i, 0)),
            pl.BlockSpec((128, LATENT), full),
            pl.BlockSpec((1, LATENT), full),
            pl.BlockSpec((LATENT, 2 * LATENT), full),
            pl.BlockSpec((4, 4 * GDIM), full),
            pl.BlockSpec((4 * GDIM, LATENT), full),
            pl.BlockSpec((4, LATENT), full),
        ],
        out_specs=[
            pl.BlockSpec((BN, LATENT), lambda i: (i, 0)),
            pl.BlockSpec((BN, LATENT), lambda i: (i, 0)),
            pl.BlockSpec((BN, LATENT), lambda i: (i, 0)),
            pl.BlockSpec((4, LATENT), full),
        ],
        out_shape=[
            jax.ShapeDtypeStruct((N_NODES, LATENT), jnp.float32),
            jax.ShapeDtypeStruct((N_NODES, LATENT), jnp.float32),
            jax.ShapeDtypeStruct((N_NODES, LATENT), jnp.float32),
            jax.ShapeDtypeStruct((4, LATENT), jnp.float32),
        ],
    )(nodes, wenc, benc, wproj, gmat, wg, brows)


def _edge_body(has_e, ce_row, gs_ref, gr_ref, e_ref, w1e_ref, w2_ref,
               b2_ref, w3_ref, b3_ref, ce_ref, out_ref):
    x = gs_ref[...] + gr_ref[...] + ce_ref[ce_row:ce_row + 1, :]
    if has_e:
        x = x + jnp.dot(e_ref[...], w1e_ref[...],
                        preferred_element_type=jnp.float32)
    h1 = _relu(x)
    h2 = _relu(jnp.dot(h1, w2_ref[...],
                       preferred_element_type=jnp.float32) + b2_ref[...])
    y = jnp.dot(h2, w3_ref[...],
                preferred_element_type=jnp.float32) + b3_ref[...]
    rows = pl.program_id(0) * BE + jax.lax.broadcasted_iota(
        jnp.int32, (BE, 1), 0)
    out_ref[...] = jnp.where(rows < N_EDGES, y, 0.0)


def _edge_call(gathered, e_prev, w1e, w2, b2, w3, b3, cvec, ce_row):
    has_e = e_prev is not None
    full = lambda i: (0, 0)
    in_specs = [
        pl.BlockSpec((BE, LATENT), lambda i: (i, 0)),
        pl.BlockSpec((BE, LATENT), lambda i: (i + NBE, 0)),
    ]
    args = [gathered, gathered]
    if has_e:
        in_specs += [pl.BlockSpec((BE, LATENT), lambda i: (i, 0)),
                     pl.BlockSpec((LATENT, LATENT), full)]
        args += [e_prev, w1e]
    in_specs += [
        pl.BlockSpec((LATENT, LATENT), full),
        pl.BlockSpec((1, LATENT), full),
        pl.BlockSpec((LATENT, LATENT), full),
        pl.BlockSpec((1, LATENT), full),
        pl.BlockSpec((4, LATENT), full),
    ]
    args += [w2, b2, w3, b3, cvec]
    body = functools.partial(_edge_body, has_e, ce_row)
    if not has_e:
        def body(gs, gr, w2r, b2r, w3r, b3r, cer, outr):
            _edge_body(False, ce_row, gs, gr, None, None, w2r, b2r, w3r,
                       b3r, cer, outr)
    return pl.pallas_call(
        body,
        grid=(NBE,),
        in_specs=in_specs,
        out_specs=pl.BlockSpec((BE, LATENT), lambda i: (i, 0)),
        out_shape=jax.ShapeDtypeStruct((E_PAD, LATENT), jnp.float32),
    )(*args)


def _node_body(final, cn_row, n_ref, r0_ref, r1_ref, a1_ref, bmat_ref,
               w2_ref, b2_ref, w3_ref, b3_ref, cn_ref, lns_ref, lnb_ref,
               wnext_ref, bnext_ref, *out_refs):
    n = n_ref[...]
    recv = r0_ref[...] + r1_ref[...]
    m1 = _relu(jnp.dot(n, a1_ref[...], preferred_element_type=jnp.float32)
               + jnp.dot(recv, bmat_ref[...],
                         preferred_element_type=jnp.float32)
               + cn_ref[cn_row:cn_row + 1, :])
    m2 = _relu(jnp.dot(m1, w2_ref[...],
                       preferred_element_type=jnp.float32) + b2_ref[...])
    nn = jnp.dot(m2, w3_ref[...],
                 preferred_element_type=jnp.float32) + b3_ref[...]
    y = _ln(nn + n, lns_ref[...], lnb_ref[...])
    if final:
        out_refs[0][...] = jnp.dot(
            y, wnext_ref[...], preferred_element_type=jnp.float32
        ) + bnext_ref[...]
    else:
        out_refs[0][...] = y
        p = jnp.dot(y, wnext_ref[...], preferred_element_type=jnp.float32)
        out_refs[1][...] = p[:, :LATENT]
        out_refs[2][...] = p[:, LATENT:]


def _node_call(final, n, r0, r1, a1, bmat, w2, b2, w3, b3, cvec, cn_row,
               lns, lnb, wnext, bnext):
    full = lambda i: (0, 0)
    blk = lambda i: (i, 0)
    next_cols = 128 if final else 2 * LATENT
    in_specs = [
        pl.BlockSpec((BN, LATENT), blk),
        pl.BlockSpec((BN, LATENT), blk),
        pl.BlockSpec((BN, LATENT), blk),
        pl.BlockSpec((LATENT, LATENT), full),
        pl.BlockSpec((LATENT, LATENT), full),
        pl.BlockSpec((LATENT, LATENT), full),
        pl.BlockSpec((1, LATENT), full),
        pl.BlockSpec((LATENT, LATENT), full),
        pl.BlockSpec((1, LATENT), full),
        pl.BlockSpec((4, LATENT), full),
        pl.BlockSpec((1, LATENT), full),
        pl.BlockSpec((1, LATENT), full),
        pl.BlockSpec((LATENT, next_cols), full),
        pl.BlockSpec((1, 128), full),
    ]
    if final:
        out_specs = pl.BlockSpec((BN, 128), blk)
        out_shape = jax.ShapeDtypeStruct((N_NODES, 128), jnp.float32)
    else:
        out_specs = [pl.BlockSpec((BN, LATENT), blk)] * 3
        out_shape = [jax.ShapeDtypeStruct((N_NODES, LATENT), jnp.float32)] * 3
    return pl.pallas_call(
        functools.partial(_node_body, final, cn_row),
        grid=(NBN,),
        in_specs=in_specs,
        out_specs=out_specs,
        out_shape=out_shape,
    )(n, r0, r1, a1, bmat, w2, b2, w3, b3, cvec, lns, lnb, wnext, bnext)


def _sc_mesh():
    return plsc.VectorSubcoreMesh(core_axis_name="c", subcore_axis_name="s",
                                  num_cores=NC, num_subcores=NS)


def _gather_rows(table, idx2):
    n_chunks = (2 * E_PAD) // (NW * CHUNK)
    n_iters = n_chunks // SLOTS

    def body(t_hbm, idx_hbm, out_hbm, *scr):
        ibufs = scr[:SLOTS]
        rbufs = scr[SLOTS:2 * SLOTS]
        sems = scr[2 * SLOTS:3 * SLOTS]
        wid = jax.lax.axis_index("s") * NC + jax.lax.axis_index("c")
        base = wid * n_chunks * CHUNK

        def it(i, carry):
            offs = [base + (i * SLOTS + k) * CHUNK for k in range(SLOTS)]
            his = [pltpu.async_copy(idx_hbm.at[pl.ds(offs[k], CHUNK)],
                                    ibufs[k], sems[k]) for k in range(SLOTS)]
            hgs = []
            for k in range(SLOTS):
                his[k].wait()
                hgs.append(pltpu.async_copy(t_hbm.at[ibufs[k]], rbufs[k],
                                            sems[k]))
            hss = []
            for k in range(SLOTS):
                hgs[k].wait()
                hss.append(pltpu.async_copy(
                    rbufs[k], out_hbm.at[pl.ds(offs[k], CHUNK)], sems[k]))
            for k in range(SLOTS):
                hss[k].wait()
            return carry

        jax.lax.fori_loop(0, n_iters, it, 0)

    scratch = ([pltpu.VMEM((CHUNK,), jnp.int32) for _ in range(SLOTS)]
               + [pltpu.VMEM((CHUNK, LATENT), jnp.float32)
                  for _ in range(SLOTS)]
               + [pltpu.SemaphoreType.DMA for _ in range(SLOTS)])
    return pl.kernel(
        body,
        out_type=jax.ShapeDtypeStruct((2 * E_PAD, LATENT), jnp.float32),
        mesh=_sc_mesh(),
        scratch_types=scratch,
        compiler_params=pltpu.CompilerParams(use_tc_tiling_on_sc=False),
    )(table, idx2)


def _segment_partials(e_new, receivers_p, zeros):
    n_chunks = E_PAD // (NW * CHUNK)
    n_iters = n_chunks // SLOTS

    def body(e_hbm, idx_hbm, z_hbm, out_hbm, acc, *scr):
        ibufs = scr[:SLOTS]
        rbufs = scr[SLOTS:2 * SLOTS]
        isems = scr[2 * SLOTS:3 * SLOTS]
        rsems = scr[3 * SLOTS:4 * SLOTS]
        cid = jax.lax.axis_index("c")
        sid = jax.lax.axis_index("s")
        wid = sid * NC + cid
        r0 = sid * ROWS_PER_TILE
        pltpu.sync_copy(z_hbm.at[pl.ds(r0, ROWS_PER_TILE)],
                        acc.at[pl.ds(r0, ROWS_PER_TILE)])
        plsc.subcore_barrier()
        base = wid * n_chunks * CHUNK

        def it(i, carry):
            offs = [base + (i * SLOTS + k) * CHUNK for k in range(SLOTS)]
            his = [pltpu.async_copy(idx_hbm.at[pl.ds(offs[k], CHUNK)],
                                    ibufs[k], isems[k]) for k in range(SLOTS)]
            hrs = [pltpu.async_copy(e_hbm.at[pl.ds(offs[k], CHUNK)],
                                    rbufs[k], rsems[k]) for k in range(SLOTS)]
            for k in range(SLOTS):
                his[k].wait()
                hrs[k].wait()
                pltpu.sync_copy(rbufs[k], acc.at[ibufs[k]], add=True)
            return carry

        jax.lax.fori_loop(0, n_iters, it, 0)
        plsc.subcore_barrier()
        pltpu.sync_copy(acc.at[pl.ds(r0, ROWS_PER_TILE)],
                        out_hbm.at[cid, pl.ds(r0, ROWS_PER_TILE)])

    scratch = ([pltpu.VMEM_SHARED((N_ACC, LATENT), jnp.float32)]
               + [pltpu.VMEM((CHUNK,), jnp.int32) for _ in range(SLOTS)]
               + [pltpu.VMEM((CHUNK, LATENT), jnp.float32)
                  for _ in range(SLOTS)]
               + [pltpu.SemaphoreType.DMA for _ in range(2 * SLOTS)])
    part = pl.kernel(
        body,
        out_type=jax.ShapeDtypeStruct((NC, N_ACC, LATENT), jnp.float32),
        mesh=_sc_mesh(),
        scratch_types=scratch,
        compiler_params=pltpu.CompilerParams(use_tc_tiling_on_sc=False),
    )(e_new, receivers_p, zeros)
    return part[0, :N_NODES], part[1, :N_NODES]


def kernel(nodes, senders, receivers, globals_, params):
    f32 = jnp.float32
    enc = params["encoder"]
    dec = params["decoder"]
    steps = params["steps"]

    pad = E_PAD - N_EDGES
    senders_p = jnp.concatenate(
        [senders, jnp.zeros((pad,), senders.dtype)]).astype(jnp.int32)
    receivers_p = jnp.concatenate(
        [receivers, jnp.zeros((pad,), receivers.dtype)]).astype(jnp.int32)
    idx2 = jnp.concatenate([senders_p, receivers_p + N_NODES])
    acczeros = jnp.zeros((N_ACC, LATENT), f32)

    g0 = globals_[0].astype(f32)

    e0W1 = steps[0]["edge_mlp"][0]["W"]
    w1s0, w1r0, w1g0 = e0W1[:64], e0W1[64:128], e0W1[128:136]
    e1W1 = steps[1]["edge_mlp"][0]["W"]
    w1e1, w1s1, w1r1, w1g1 = (e1W1[:64], e1W1[64:128], e1W1[128:192],
                              e1W1[192:200])
    n0W1 = steps[0]["node_mlp"][0]["W"]
    a10, bm0, wgn0 = n0W1[:64], n0W1[64:128], n0W1[128:136]
    n1W1 = steps[1]["node_mlp"][0]["W"]
    a11, bm1, wgn1 = n1W1[:64], n1W1[64:128], n1W1[128:136]

    gmat = jnp.zeros((4, 4 * GDIM), f32)
    gmat = gmat.at[0, 0:8].set(g0)
    gmat = gmat.at[1, 8:16].set(g0)
    gmat = gmat.at[2, 16:24].set(2.0 * g0)
    gmat = gmat.at[3, 24:32].set(2.0 * g0)
    wg_all = jnp.concatenate([w1g0, wgn0, w1g1, wgn1], axis=0)
    brows = jnp.stack([
        steps[0]["edge_mlp"][0]["b"], steps[0]["node_mlp"][0]["b"],
        steps[1]["edge_mlp"][0]["b"], steps[1]["node_mlp"][0]["b"]])

    row = lambda b: b.reshape(1, -1)
    wproj0 = jnp.concatenate([w1s0, w1r0], axis=1)
    wproj1 = jnp.concatenate([w1s1, w1r1], axis=1)

    n0, ps0, pr0, cvec = _encoder_call(
        nodes, enc["W"], row(enc["b"]), wproj0, gmat, wg_all, brows)

    t0 = jnp.concatenate([ps0, pr0], axis=0)
    g0rows = _gather_rows(t0, idx2)
    em0 = steps[0]["edge_mlp"]
    e_new0 = _edge_call(g0rows, None, None, em0[1]["W"], row(em0[1]["b"]),
                        em0[2]["W"], row(em0[2]["b"]), cvec, 0)
    p0a, p0b = _segment_partials(e_new0, receivers_p, acczeros)
    nm0 = steps[0]["node_mlp"]
    n1, ps1, pr1 = _node_call(
        False, n0, p0a, p0b, a10, bm0, nm0[1]["W"], row(nm0[1]["b"]),
        nm0[2]["W"], row(nm0[2]["b"]), cvec, 1,
        row(steps[0]["ln_scale"]), row(steps[0]["ln_bias"]),
        wproj1, jnp.zeros((1, 128), f32))

    t1 = jnp.concatenate([ps1, pr1], axis=0)
    g1rows = _gather_rows(t1, idx2)
    em1 = steps[1]["edge_mlp"]
    e_new1 = _edge_call(g1rows, e_new0, w1e1, em1[1]["W"], row(em1[1]["b"]),
                        em1[2]["W"], row(em1[2]["b"]), cvec, 2)
    p1a, p1b = _segment_partials(e_new1, receivers_p, acczeros)
    nm1 = steps[1]["node_mlp"]
    out = _node_call(
        True, n1, p1a, p1b, a11, bm1, nm1[1]["W"], row(nm1[1]["b"]),
        nm1[2]["W"], row(nm1[2]["b"]), cvec, 3,
        row(steps[1]["ln_scale"]), row(steps[1]["ln_bias"]),
        dec["W"], row(dec["b"]))
    return out
